# baseline (speedup 1.0000x reference)
import functools

import jax
import jax.numpy as jnp
from jax import lax
from jax.experimental import pallas as pl
from jax.experimental.pallas import tpu as pltpu
from jax.experimental.pallas import tpu_sc as plsc

B = 4096
V = 100000
D = 32
T = 26
NUM = 13

NC, NS = 2, 16
NW = NC * NS
BPW = B // NW


@functools.lru_cache(maxsize=None)
def _make_sc_gather():
    mesh = plsc.VectorSubcoreMesh(
        core_axis_name="c", subcore_axis_name="s",
        num_cores=NC, num_subcores=NS)

    @functools.partial(
        pl.kernel,
        out_type=jax.ShapeDtypeStruct((NW, T, BPW, D), jnp.float32),
        mesh=mesh,
        scratch_types=[
            pltpu.VMEM((T, BPW), jnp.int32),
            pltpu.VMEM((T, BPW, D), jnp.float32),
            pltpu.SemaphoreType.DMA,
        ],
        compiler_params=pltpu.CompilerParams(use_tc_tiling_on_sc=False),
    )
    def _sc_gather(table_hbm, gidx_hbm, out_hbm, idx_v, rows_v, sem):
        wid = lax.axis_index("s") * NC + lax.axis_index("c")
        pltpu.sync_copy(gidx_hbm.at[wid], idx_v)
        copies = [
            pltpu.async_copy(
                table_hbm.at[t].at[idx_v.at[t]],
                rows_v.at[t],
                sem,
            )
            for t in range(T)
        ]
        for c in copies:
            c.wait()
        pltpu.sync_copy(rows_v, out_hbm.at[wid])

    return _sc_gather


def _mlp_body(emb_ref, num_ref, w1e_ref, w1n_ref, b1_ref, w2_ref, b2_ref,
              w3_ref, b3_ref, w4_ref, b4_ref, out_ref):
    x = jnp.concatenate([emb_ref[0, t] for t in range(T)], axis=-1)
    x1 = jnp.dot(x, w1e_ref[...], preferred_element_type=jnp.float32)
    x1 = x1 + jnp.dot(num_ref[...], w1n_ref[...],
                      preferred_element_type=jnp.float32)
    h = jnp.maximum(x1 + b1_ref[...], 0.0)
    h = jnp.maximum(
        jnp.dot(h, w2_ref[...], preferred_element_type=jnp.float32)
        + b2_ref[...], 0.0)
    h = jnp.maximum(
        jnp.dot(h, w3_ref[...], preferred_element_type=jnp.float32)
        + b3_ref[...], 0.0)
    out_ref[...] = (
        jnp.dot(h, w4_ref[...], preferred_element_type=jnp.float32)
        + b4_ref[...])


def _mlp(emb4, num, w1e, w1n, b1, w2, b2, w3, b3, w4, b4, *, interpret=False):
    full = lambda shape: pl.BlockSpec(shape, lambda i: (0,) * len(shape))
    return pl.pallas_call(
        _mlp_body,
        grid=(NW,),
        in_specs=[
            pl.BlockSpec((1, T, BPW, D), lambda i: (i, 0, 0, 0)),
            pl.BlockSpec((BPW, NUM), lambda i: (i, 0)),
            full((T * D, 512)),
            full((NUM, 512)),
            full((1, 512)),
            full((512, 256)),
            full((1, 256)),
            full((256, 128)),
            full((1, 128)),
            full((128, 1)),
            full((1, 1)),
        ],
        out_specs=pl.BlockSpec((BPW, 1), lambda i: (i, 0)),
        out_shape=jax.ShapeDtypeStruct((B, 1), jnp.float32),
        interpret=interpret,
    )(emb4, num, w1e, w1n, b1, w2, b2, w3, b3, w4, b4)


def kernel(numerical_features, cat_features, tables, W1, b1, W2, b2, W3, b3,
           W4, b4):
    cat = cat_features.reshape(T, B).astype(jnp.int32)
    gidx = cat.reshape(T, NW, BPW).transpose(1, 0, 2)

    emb4 = _make_sc_gather()(tables, gidx)

    w1e = W1[: T * D]
    w1n = W1[T * D:]
    return _mlp(emb4, numerical_features, w1e, w1n, b1.reshape(1, 512),
                W2, b2.reshape(1, 256), W3, b3.reshape(1, 128),
                W4, b4.reshape(1, 1))


# speedup vs baseline: 1.8258x; 1.8258x over previous
import functools

import jax
import jax.numpy as jnp
from jax import lax
from jax.experimental import pallas as pl
from jax.experimental.pallas import tpu as pltpu
from jax.experimental.pallas import tpu_sc as plsc

B = 4096
V = 100000
D = 32
T = 26
NUM = 13

NC, NS = 2, 16
NW = NC * NS
BPW = B // NW
LANES = 16


@functools.lru_cache(maxsize=None)
def _make_sc_gather():
    mesh = plsc.VectorSubcoreMesh(
        core_axis_name="c", subcore_axis_name="s",
        num_cores=NC, num_subcores=NS)

    @functools.partial(
        pl.kernel,
        out_type=jax.ShapeDtypeStruct((NW, T, D, BPW), jnp.float32),
        mesh=mesh,
        scratch_types=[
            pltpu.VMEM((T, BPW), jnp.int32),
            pltpu.VMEM((D, BPW), jnp.int32),
            pltpu.VMEM((D, BPW), jnp.float32),
            pltpu.SemaphoreType.DMA,
        ],
        compiler_params=pltpu.CompilerParams(use_tc_tiling_on_sc=False),
    )
    def _sc_gather(flat_hbm, gidx_hbm, out_hbm, idx_v, fidx_v, rows_v, sem):
        wid = lax.axis_index("s") * NC + lax.axis_index("c")
        pltpu.sync_copy(gidx_hbm.at[wid], idx_v)

        def per_table(t, _):
            for d in range(D):
                base = (t * D + d) * V
                for c in range(BPW // LANES):
                    fidx_v[d, pl.ds(c * LANES, LANES)] = (
                        idx_v[t, pl.ds(c * LANES, LANES)] + base)
            copies = [
                pltpu.async_copy(
                    flat_hbm.at[fidx_v.at[d]], rows_v.at[d], sem)
                for d in range(D)
            ]
            for cp in copies:
                cp.wait()
            pltpu.sync_copy(rows_v, out_hbm.at[wid, t])
            return ()

        lax.fori_loop(0, T, per_table, (), unroll=False)

    return _sc_gather


def _mlp_body(emb_ref, num_ref, w1e_ref, w1n_ref, b1_ref, w2_ref, b2_ref,
              w3_ref, b3_ref, w4_ref, b4_ref, out_ref):
    kt = emb_ref[0].reshape(T * D, BPW)
    x1 = lax.dot_general(kt, w1e_ref[...], (((0,), (0,)), ((), ())),
                         preferred_element_type=jnp.float32)
    x1 = x1 + jnp.dot(num_ref[...], w1n_ref[...],
                      preferred_element_type=jnp.float32)
    h = jnp.maximum(x1 + b1_ref[...], 0.0)
    h = jnp.maximum(
        jnp.dot(h, w2_ref[...], preferred_element_type=jnp.float32)
        + b2_ref[...], 0.0)
    h = jnp.maximum(
        jnp.dot(h, w3_ref[...], preferred_element_type=jnp.float32)
        + b3_ref[...], 0.0)
    out_ref[...] = (
        jnp.dot(h, w4_ref[...], preferred_element_type=jnp.float32)
        + b4_ref[...])


def _mlp(emb5, num, w1e, w1n, b1, w2, b2, w3, b3, w4, b4, *, interpret=False):
    full = lambda shape: pl.BlockSpec(shape, lambda i: (0,) * len(shape))
    return pl.pallas_call(
        _mlp_body,
        grid=(NW,),
        in_specs=[
            pl.BlockSpec((1, T, D, BPW), lambda i: (i, 0, 0, 0)),
            pl.BlockSpec((BPW, NUM), lambda i: (i, 0)),
            full((T * D, 512)),
            full((NUM, 512)),
            full((1, 512)),
            full((512, 256)),
            full((1, 256)),
            full((256, 128)),
            full((1, 128)),
            full((128, 1)),
            full((1, 1)),
        ],
        out_specs=pl.BlockSpec((BPW, 1), lambda i: (i, 0)),
        out_shape=jax.ShapeDtypeStruct((B, 1), jnp.float32),
        interpret=interpret,
    )(emb5, num, w1e, w1n, b1, w2, b2, w3, b3, w4, b4)


def kernel(numerical_features, cat_features, tables, W1, b1, W2, b2, W3, b3,
           W4, b4):
    flat = tables.transpose(0, 2, 1).reshape(T * D * V)

    cat = cat_features.reshape(T, B).astype(jnp.int32)
    gidx = cat.reshape(T, NW, BPW).transpose(1, 0, 2)

    emb5 = _make_sc_gather()(flat, gidx)

    w1e = W1[: T * D]
    w1n = W1[T * D:]
    return _mlp(emb5, numerical_features, w1e, w1n, b1.reshape(1, 512),
                W2, b2.reshape(1, 256), W3, b3.reshape(1, 128),
                W4, b4.reshape(1, 1))


# speedup vs baseline: 2.8897x; 1.5827x over previous
import functools

import jax
import jax.numpy as jnp
from jax import lax
from jax.experimental import pallas as pl
from jax.experimental.pallas import tpu as pltpu
from jax.experimental.pallas import tpu_sc as plsc

B = 4096
V = 100000
D = 32
T = 26
NUM = 13

NC, NS = 2, 16
NW = NC * NS
BPW = B // NW
LANES = 16


@functools.lru_cache(maxsize=None)
def _make_sc_gather():
    mesh = plsc.VectorSubcoreMesh(
        core_axis_name="c", subcore_axis_name="s",
        num_cores=NC, num_subcores=NS)

    @functools.partial(
        pl.kernel,
        out_type=jax.ShapeDtypeStruct((NW, T, D, BPW), jnp.float32),
        mesh=mesh,
        scratch_types=[
            pltpu.VMEM((T, BPW), jnp.int32),
            pltpu.VMEM((D, BPW), jnp.int32),
            pltpu.VMEM((D, BPW), jnp.float32),
            pltpu.SemaphoreType.DMA,
        ],
        compiler_params=pltpu.CompilerParams(use_tc_tiling_on_sc=False),
    )
    def _sc_gather(flat_hbm, gidx_hbm, out_hbm, idx_v, fidx_v, rows_v, sem):
        wid = lax.axis_index("s") * NC + lax.axis_index("c")
        pltpu.sync_copy(gidx_hbm.at[wid], idx_v)

        def per_table(t, _):
            for d in range(D):
                base = (t * D + d) * V
                for c in range(BPW // LANES):
                    fidx_v[d, pl.ds(c * LANES, LANES)] = (
                        idx_v[t, pl.ds(c * LANES, LANES)] + base)
            copies = [
                pltpu.async_copy(
                    flat_hbm.at[fidx_v.at[d]], rows_v.at[d], sem)
                for d in range(D)
            ]
            for cp in copies:
                cp.wait()
            pltpu.sync_copy(rows_v, out_hbm.at[wid, t])
            return ()

        lax.fori_loop(0, T, per_table, (), unroll=False)

    return _sc_gather


def _detile_body(tab_ref, out_ref):
    for k in range(D):
        out_ref[pl.ds(k * V, V)] = tab_ref[0, k, :]


def _detile(tab_t):
    return pl.pallas_call(
        _detile_body,
        grid=(T,),
        in_specs=[pl.BlockSpec((1, D, V), lambda t: (t, 0, 0))],
        out_specs=pl.BlockSpec((D * V,), lambda t: (t,)),
        out_shape=jax.ShapeDtypeStruct((T * D * V,), jnp.float32),
        compiler_params=pltpu.CompilerParams(
            vmem_limit_bytes=120 * 1024 * 1024),
    )(tab_t)


def _mlp_body(emb_ref, num_ref, w1e_ref, w1n_ref, b1_ref, w2_ref, b2_ref,
              w3_ref, b3_ref, w4_ref, b4_ref, out_ref):
    kt = emb_ref[0].reshape(T * D, BPW)
    x1 = lax.dot_general(kt, w1e_ref[...], (((0,), (0,)), ((), ())),
                         preferred_element_type=jnp.float32)
    x1 = x1 + jnp.dot(num_ref[...], w1n_ref[...],
                      preferred_element_type=jnp.float32)
    h = jnp.maximum(x1 + b1_ref[...], 0.0)
    h = jnp.maximum(
        jnp.dot(h, w2_ref[...], preferred_element_type=jnp.float32)
        + b2_ref[...], 0.0)
    h = jnp.maximum(
        jnp.dot(h, w3_ref[...], preferred_element_type=jnp.float32)
        + b3_ref[...], 0.0)
    out_ref[...] = (
        jnp.dot(h, w4_ref[...], preferred_element_type=jnp.float32)
        + b4_ref[...])


def _mlp(emb5, num, w1e, w1n, b1, w2, b2, w3, b3, w4, b4, *, interpret=False):
    full = lambda shape: pl.BlockSpec(shape, lambda i: (0,) * len(shape))
    return pl.pallas_call(
        _mlp_body,
        grid=(NW,),
        in_specs=[
            pl.BlockSpec((1, T, D, BPW), lambda i: (i, 0, 0, 0)),
            pl.BlockSpec((BPW, NUM), lambda i: (i, 0)),
            full((T * D, 512)),
            full((NUM, 512)),
            full((1, 512)),
            full((512, 256)),
            full((1, 256)),
            full((256, 128)),
            full((1, 128)),
            full((128, 1)),
            full((1, 1)),
        ],
        out_specs=pl.BlockSpec((BPW, 1), lambda i: (i, 0)),
        out_shape=jax.ShapeDtypeStruct((B, 1), jnp.float32),
        interpret=interpret,
    )(emb5, num, w1e, w1n, b1, w2, b2, w3, b3, w4, b4)


def kernel(numerical_features, cat_features, tables, W1, b1, W2, b2, W3, b3,
           W4, b4):
    flat = _detile(tables.transpose(0, 2, 1))

    cat = cat_features.reshape(T, B).astype(jnp.int32)
    gidx = cat.reshape(T, NW, BPW).transpose(1, 0, 2)

    emb5 = _make_sc_gather()(flat, gidx)

    w1e = W1[: T * D]
    w1n = W1[T * D:]
    return _mlp(emb5, numerical_features, w1e, w1n, b1.reshape(1, 512),
                W2, b2.reshape(1, 256), W3, b3.reshape(1, 128),
                W4, b4.reshape(1, 1))
